# baseline (speedup 1.0000x reference)
import functools

import jax
import jax.numpy as jnp
from jax import lax
from jax.experimental import pallas as pl
from jax.experimental.pallas import tpu as pltpu
from jax.experimental.pallas import tpu_sc as plsc

V = 1_000_000
VP = 1_000_064
L_TOKENS = 16384
EMBED = 64
OUT = 128

NC = 2
NS = 16
NW = NC * NS
PER_W = L_TOKENS // NW
ISZ = 128
NI = PER_W // ISZ

CH = 16384
NCH = (VP + CH - 1) // CH

BLK = 32768
TAILB = 30
G1 = 15
S_SC = G1 * BLK
R_SC = TAILB * BLK - S_SC

NROW = 16
NRG = EMBED // NROW
NCG = NW // NRG
COLS_T = R_SC // NCG
CHK = 2048
NCHK = COLS_T // CHK


def _sc_counts(idx):
    mesh = plsc.VectorSubcoreMesh(core_axis_name="c", subcore_axis_name="s")

    @functools.partial(
        pl.kernel,
        mesh=mesh,
        out_type=jax.ShapeDtypeStruct((NC, VP), jnp.float32),
        scratch_types=[
            pltpu.VMEM((NI, ISZ), jnp.int32),
            pltpu.VMEM((CH,), jnp.float32),
            pltpu.VMEM((ISZ,), jnp.float32),
            pltpu.VMEM_SHARED((VP,), jnp.float32),
            pltpu.SemaphoreType.DMA,
        ],
    )
    def k(idx_hbm, out_hbm, idx_v, z_v, one_v, c_sh, sem):
        cid = lax.axis_index("c")
        sid = lax.axis_index("s")
        wid = cid * NS + sid

        zero = jnp.zeros((16,), jnp.float32)
        for t in range(CH // 16):
            z_v[pl.ds(t * 16, 16)] = zero
        one = jnp.full((16,), 1.0, jnp.float32)
        for t in range(ISZ // 16):
            one_v[pl.ds(t * 16, 16)] = one

        for t in range(NCH):
            ln = CH if t < NCH - 1 else VP - (NCH - 1) * CH

            @pl.when(sid == (t % NS))
            def _zero(t=t, ln=ln):
                pltpu.sync_copy(z_v.at[pl.ds(0, ln)], c_sh.at[pl.ds(t * CH, ln)])

        base = wid * PER_W
        for j in range(NI):
            pltpu.sync_copy(idx_hbm.at[pl.ds(base + j * ISZ, ISZ)], idx_v.at[j])
        plsc.subcore_barrier()

        copies = [
            pltpu.async_copy(one_v, c_sh.at[idx_v.at[j]], sem, add=True)
            for j in range(NI)
        ]
        for cp in copies:
            cp.wait()
        plsc.subcore_barrier()

        for t in range(NCH):
            ln = CH if t < NCH - 1 else VP - (NCH - 1) * CH

            @pl.when(sid == (t % NS))
            def _out(t=t, ln=ln):
                pltpu.sync_copy(
                    c_sh.at[pl.ds(t * CH, ln)],
                    out_hbm.at[cid, pl.ds(t * CH, ln)],
                )

    return k(idx)


def _sc_scan(table_t, counts):
    mesh = plsc.VectorSubcoreMesh(core_axis_name="c", subcore_axis_name="s")

    @functools.partial(
        pl.kernel,
        mesh=mesh,
        out_type=jax.ShapeDtypeStruct((1, NW * NROW * 16), jnp.float32),
        scratch_types=[
            pltpu.VMEM((2, NROW, CHK), jnp.float32),
            pltpu.VMEM((2, NC, CHK), jnp.float32),
            pltpu.VMEM((CHK,), jnp.float32),
            pltpu.VMEM((NROW * 16,), jnp.float32),
            pltpu.SemaphoreType.DMA,
            pltpu.SemaphoreType.DMA,
        ],
    )
    def k(tbl_hbm, cnt_hbm, out_hbm, t_v, c_v, cs_v, acc_v, tsem, csem):
        cid = lax.axis_index("c")
        sid = lax.axis_index("s")
        wid = cid * NS + sid
        rg = lax.rem(wid, NRG)
        cg = lax.div(wid, NRG)
        row0 = pl.multiple_of(rg * NROW, NROW)
        cbase = S_SC + cg * COLS_T

        zero = jnp.zeros((16,), jnp.float32)
        for e in range(NROW):
            acc_v[pl.ds(e * 16, 16)] = zero

        def issue(i, b):
            col = pl.multiple_of(cbase + i * CHK, 128)
            pltpu.async_copy(
                tbl_hbm.at[pl.ds(row0, NROW), pl.ds(col, CHK)], t_v.at[b], tsem
            )
            pltpu.async_copy(cnt_hbm.at[0, pl.ds(col, CHK)], c_v.at[b, 0], csem)
            pltpu.async_copy(cnt_hbm.at[1, pl.ds(col, CHK)], c_v.at[b, 1], csem)

        issue(0, 0)

        def chunk(i, _):
            b = lax.rem(i, 2)

            @pl.when(i + 1 < NCHK)
            def _next():
                issue(i + 1, 1 - b)

            pltpu.make_async_copy(
                tbl_hbm.at[pl.ds(0, NROW), pl.ds(0, CHK)], t_v.at[b], tsem
            ).wait()
            pltpu.make_async_copy(
                cnt_hbm.at[pl.ds(0, NC), pl.ds(0, CHK)], c_v.at[b], csem
            ).wait()

            for q in range(CHK // 16):
                cs_v[pl.ds(q * 16, 16)] = (
                    c_v[b, 0, pl.ds(q * 16, 16)] + c_v[b, 1, pl.ds(q * 16, 16)]
                )
            for e in range(NROW):
                parts = []
                for pp in range(4):
                    q0 = pp * (CHK // 64)
                    s = t_v[b, e, pl.ds(q0 * 16, 16)] * cs_v[pl.ds(q0 * 16, 16)]
                    for q in range(q0 + 1, q0 + CHK // 64):
                        s = s + t_v[b, e, pl.ds(q * 16, 16)] * cs_v[pl.ds(q * 16, 16)]
                    parts.append(s)
                acc_v[pl.ds(e * 16, 16)] = acc_v[pl.ds(e * 16, 16)] + (
                    (parts[0] + parts[1]) + (parts[2] + parts[3])
                )
            return 0

        lax.fori_loop(0, NCHK, chunk, 0)
        pltpu.sync_copy(acc_v, out_hbm.at[0, pl.ds(wid * NROW * 16, NROW * 16)])

    return k(table_t, counts)


def _tc_scan(table_t, counts):
    def k(t_ref, c_ref, o_ref):
        g = pl.program_id(0)

        @pl.when(g == 0)
        def _init():
            o_ref[...] = jnp.zeros_like(o_ref)

        blk = jnp.where(g < G1, g, TAILB)
        col = blk * BLK + lax.broadcasted_iota(jnp.int32, (1, BLK), 1)
        valid = col < V
        c = jnp.where(valid, (c_ref[0, :] + c_ref[1, :])[None, :], 0.0)
        t = jnp.where(valid, t_ref[...], 0.0)
        o_ref[...] += lax.dot_general(
            c, t, (((1,), (1,)), ((), ())),
            preferred_element_type=jnp.float32,
        )

    bmap = lambda g: (0, jnp.where(g < G1, g, TAILB))
    return pl.pallas_call(
        k,
        grid=(G1 + 1,),
        in_specs=[
            pl.BlockSpec((EMBED, BLK), bmap),
            pl.BlockSpec((NC, BLK), bmap),
        ],
        out_specs=pl.BlockSpec((1, EMBED), lambda g: (0, 0)),
        out_shape=jax.ShapeDtypeStruct((1, EMBED), jnp.float32),
    )(table_t, counts)


def _tc_tail(tc_emb, sc_parts, w1, b1):
    NP = NW * NROW * 16

    def k(e_ref, p_ref, w_ref, b_ref, o_ref):
        i = lax.broadcasted_iota(jnp.int32, (NP, EMBED), 0)
        e = lax.broadcasted_iota(jnp.int32, (NP, EMBED), 1)
        src_e = lax.rem(i // (NROW * 16), NRG) * NROW + lax.rem(i, NROW * 16) // 16
        sel = jnp.where(src_e == e, 1.0, 0.0).astype(jnp.float32)
        emb = e_ref[...] + lax.dot_general(
            p_ref[...], sel, (((1,), (0,)), ((), ())),
            preferred_element_type=jnp.float32,
        )
        o_ref[...] = (
            lax.dot_general(
                emb,
                w_ref[...],
                (((1,), (1,)), ((), ())),
                preferred_element_type=jnp.float32,
            )
            + b_ref[...]
        )

    return pl.pallas_call(
        k,
        out_shape=jax.ShapeDtypeStruct((1, OUT), jnp.float32),
    )(tc_emb, sc_parts, w1, b1)


@jax.jit
def kernel(inputs, embeddings, W1, b1):
    idx = inputs.astype(jnp.int32)
    counts = _sc_counts(idx)
    table_t = embeddings.T
    sc_parts = _sc_scan(table_t, counts)
    tc_emb = _tc_scan(table_t, counts)
    return _tc_tail(tc_emb, sc_parts, W1, b1.reshape(1, OUT))


# speedup vs baseline: 3.7316x; 3.7316x over previous
import functools

import jax
import jax.numpy as jnp
from jax import lax
from jax.experimental import pallas as pl
from jax.experimental.pallas import tpu as pltpu
from jax.experimental.pallas import tpu_sc as plsc

V = 1_000_000
VP = 1_000_064
L_TOKENS = 16384
EMBED = 64
OUT = 128

NC = 2
NS = 16
NW = NC * NS
PER_W = L_TOKENS // NW
ISZ = 128
NI = PER_W // ISZ

CH = 16384
NCH = (VP + CH - 1) // CH

BLK = 32768
GRID = (V + BLK - 1) // BLK


def _sc_counts(idx):
    mesh = plsc.VectorSubcoreMesh(core_axis_name="c", subcore_axis_name="s")

    @functools.partial(
        pl.kernel,
        mesh=mesh,
        out_type=jax.ShapeDtypeStruct((NC, VP), jnp.float32),
        scratch_types=[
            pltpu.VMEM((NI, ISZ), jnp.int32),
            pltpu.VMEM((CH,), jnp.float32),
            pltpu.VMEM((ISZ,), jnp.float32),
            pltpu.VMEM_SHARED((VP,), jnp.float32),
            pltpu.SemaphoreType.DMA,
        ],
    )
    def k(idx_hbm, out_hbm, idx_v, z_v, one_v, c_sh, sem):
        cid = lax.axis_index("c")
        sid = lax.axis_index("s")
        wid = cid * NS + sid

        zero = jnp.zeros((16,), jnp.float32)
        for t in range(CH // 16):
            z_v[pl.ds(t * 16, 16)] = zero
        one = jnp.full((16,), 1.0, jnp.float32)
        for t in range(ISZ // 16):
            one_v[pl.ds(t * 16, 16)] = one

        TAILW = VP - (NCH - 1) * CH
        zc = [
            pltpu.async_copy(
                z_v, c_sh.at[pl.ds((u * NS + sid) * CH, CH)], sem
            )
            for u in range(3)
        ]

        @pl.when(sid < NCH - 1 - 3 * NS)
        def _zfull():
            pltpu.async_copy(
                z_v, c_sh.at[pl.ds((3 * NS + sid) * CH, CH)], sem
            ).wait()

        @pl.when(sid == NCH - 1 - 3 * NS)
        def _ztail():
            pltpu.async_copy(
                z_v.at[pl.ds(0, TAILW)],
                c_sh.at[pl.ds((NCH - 1) * CH, TAILW)],
                sem,
            ).wait()

        base = wid * PER_W
        for j in range(NI):
            pltpu.sync_copy(idx_hbm.at[pl.ds(base + j * ISZ, ISZ)], idx_v.at[j])
        for c in zc:
            c.wait()
        plsc.subcore_barrier()

        copies = [
            pltpu.async_copy(one_v, c_sh.at[idx_v.at[j]], sem, add=True)
            for j in range(NI)
        ]
        for cp in copies:
            cp.wait()
        plsc.subcore_barrier()

        oc = [
            pltpu.async_copy(
                c_sh.at[pl.ds((u * NS + sid) * CH, CH)],
                out_hbm.at[cid, pl.ds((u * NS + sid) * CH, CH)],
                sem,
            )
            for u in range(3)
        ]

        @pl.when(sid < NCH - 1 - 3 * NS)
        def _ofull():
            pltpu.async_copy(
                c_sh.at[pl.ds((3 * NS + sid) * CH, CH)],
                out_hbm.at[cid, pl.ds((3 * NS + sid) * CH, CH)],
                sem,
            ).wait()

        @pl.when(sid == NCH - 1 - 3 * NS)
        def _otail():
            pltpu.async_copy(
                c_sh.at[pl.ds((NCH - 1) * CH, TAILW)],
                out_hbm.at[cid, pl.ds((NCH - 1) * CH, TAILW)],
                sem,
            ).wait()

        for c in oc:
            c.wait()

    return k(idx)


def _tc_scan_tail(table_t, counts, w1, b1):
    def k(t_ref, c_ref, w_ref, b_ref, o_ref, acc_ref):
        g = pl.program_id(0)

        @pl.when(g == 0)
        def _init():
            acc_ref[...] = jnp.zeros_like(acc_ref)

        col = g * BLK + lax.broadcasted_iota(jnp.int32, (1, BLK), 1)
        valid = col < V
        c = jnp.where(valid, (c_ref[0, :] + c_ref[1, :])[None, :], 0.0)
        t = jnp.where(valid, t_ref[...], 0.0)
        acc_ref[...] += lax.dot_general(
            c, t, (((1,), (1,)), ((), ())),
            preferred_element_type=jnp.float32,
        )

        @pl.when(g == GRID - 1)
        def _tail():
            o_ref[...] = (
                lax.dot_general(
                    acc_ref[...],
                    w_ref[...],
                    (((1,), (1,)), ((), ())),
                    preferred_element_type=jnp.float32,
                )
                + b_ref[...]
            )

    return pl.pallas_call(
        k,
        grid=(GRID,),
        in_specs=[
            pl.BlockSpec((EMBED, BLK), lambda g: (0, g)),
            pl.BlockSpec((NC, BLK), lambda g: (0, g)),
            pl.BlockSpec((OUT, EMBED), lambda g: (0, 0)),
            pl.BlockSpec((1, OUT), lambda g: (0, 0)),
        ],
        out_specs=pl.BlockSpec((1, OUT), lambda g: (0, 0)),
        scratch_shapes=[pltpu.VMEM((1, EMBED), jnp.float32)],
        out_shape=jax.ShapeDtypeStruct((1, OUT), jnp.float32),
    )(table_t, counts, w1, b1)


@jax.jit
def kernel(inputs, embeddings, W1, b1):
    idx = inputs.astype(jnp.int32)
    counts = _sc_counts(idx)
    return _tc_scan_tail(embeddings.T, counts, W1, b1.reshape(1, OUT))
